# Initial kernel scaffold; baseline (speedup 1.0000x reference)
import jax
import jax.numpy as jnp
from jax.experimental import pallas as pl


def kernel(logits, labels):
    raise NotImplementedError("write your pallas kernel here")


# speedup vs baseline: 2.5087x; 2.5087x over previous
import math
import functools

import jax
import jax.numpy as jnp
from jax.experimental import pallas as pl

S = 64.0
MARGIN = 0.5
COS_M = math.cos(MARGIN)
SIN_M = math.sin(MARGIN)

ROW_BLOCK = 256
COL_BLOCK = 2048


def _arcface_kernel(labels_ref, x_ref, o_ref):
    j = pl.program_id(1)
    x = x_ref[...]
    lab = labels_ref[...]
    cols = jax.lax.broadcasted_iota(jnp.int32, x.shape, 1) + j * COL_BLOCK
    scaled = x * S
    fixed = x * (S * COS_M) - jnp.sqrt(jnp.maximum(1.0 - x * x, 0.0)) * (S * SIN_M)
    o_ref[...] = jnp.where(lab == cols, fixed, scaled)


@jax.jit
def kernel(logits, labels):
    n_rows, n_cols = logits.shape
    labels2d = labels.reshape(n_rows, 1)
    grid = (n_rows // ROW_BLOCK, pl.cdiv(n_cols, COL_BLOCK))
    return pl.pallas_call(
        _arcface_kernel,
        grid=grid,
        in_specs=[
            pl.BlockSpec((ROW_BLOCK, 1), lambda i, j: (i, 0)),
            pl.BlockSpec((ROW_BLOCK, COL_BLOCK), lambda i, j: (i, j)),
        ],
        out_specs=pl.BlockSpec((ROW_BLOCK, COL_BLOCK), lambda i, j: (i, j)),
        out_shape=jax.ShapeDtypeStruct((n_rows, n_cols), logits.dtype),
    )(labels2d, logits)
